# baseline (speedup 1.0000x reference)
import functools

import jax
import jax.numpy as jnp
from jax.experimental import pallas as pl
from jax.experimental.pallas import tpu as pltpu


def _body(x_ref, t_ref, o_ref, *, scale):
    o_ref[...] = x_ref[...] + scale * t_ref[...][None]


def kernel(inputs, lookup_table):
    B, T, D = inputs.shape
    scale = float(D) ** 0.5

    BB = 128
    grid = (B // BB,)

    out = pl.pallas_call(
        functools.partial(_body, scale=scale),
        grid=grid,
        in_specs=[
            pl.BlockSpec((BB, T, D), lambda i: (i, 0, 0)),
            pl.BlockSpec((T, D), lambda i: (0, 0)),
        ],
        out_specs=pl.BlockSpec((BB, T, D), lambda i: (i, 0, 0)),
        out_shape=jax.ShapeDtypeStruct((B, T, D), jnp.float32),
    )(inputs, lookup_table)
    return out


# speedup vs baseline: 1.6705x; 1.6705x over previous
import functools

import jax
import jax.numpy as jnp
from jax.experimental import pallas as pl
from jax.experimental.pallas import tpu as pltpu


def _body(x_ref, t_ref, o_ref, *, scale):
    o_ref[...] = x_ref[...] + scale * t_ref[...]


def kernel(inputs, lookup_table):
    B, T, D = inputs.shape
    scale = float(D) ** 0.5
    TD = T * D
    x = inputs.reshape(B, TD)
    tbl = lookup_table.reshape(1, TD)

    BB = 256
    grid = (B // BB,)

    out = pl.pallas_call(
        functools.partial(_body, scale=scale),
        grid=grid,
        in_specs=[
            pl.BlockSpec((BB, TD), lambda i: (i, 0)),
            pl.BlockSpec((1, TD), lambda i: (0, 0)),
        ],
        out_specs=pl.BlockSpec((BB, TD), lambda i: (i, 0)),
        out_shape=jax.ShapeDtypeStruct((B, TD), jnp.float32),
    )(x, tbl)
    return out.reshape(B, T, D)
